# baseline (speedup 1.0000x reference)
import functools

import jax
import jax.numpy as jnp
from jax import lax
from jax.experimental import pallas as pl
from jax.experimental.pallas import tpu as pltpu
from jax.experimental.pallas import tpu_sc as plsc

T = 2048
D = 1024
E = 8
F = 512
SF = 512
NG = 4
GSZ = E // NG
SCALE = 2.446
A = 2 * T

BT = 1024
NB = A // BT
G = NB + E

NC = 2
NS = 16
NW = NC * NS
TPW = T // NW
CH = 16


def _router_body(x_ref, gw_ref, gb_ref, r0_ref, r1_ref, w0_ref, w1_ref,
                 te_ref, rb_ref, lo_ref, hi_ref):
    x = x_ref[...]
    gates = lax.dot_general(x, gw_ref[...], (((1,), (1,)), ((), ())),
                            preferred_element_type=jnp.float32)
    sig = jax.nn.sigmoid(gates)
    s = sig + gb_ref[...]

    gsum = jnp.concatenate(
        [s[:, g * GSZ:g * GSZ + 1] + s[:, g * GSZ + 1:g * GSZ + 2]
         for g in range(NG)], axis=1)

    neg = jnp.float32(-jnp.inf)
    idx4 = lax.broadcasted_iota(jnp.int32, (T, NG), 1)
    m1 = jnp.max(gsum, axis=1, keepdims=True)
    g1 = jnp.min(jnp.where(gsum == m1, idx4, NG), axis=1, keepdims=True)
    gs2 = jnp.where(idx4 == g1, neg, gsum)
    m2 = jnp.max(gs2, axis=1, keepdims=True)
    g2 = jnp.min(jnp.where(gs2 == m2, idx4, NG), axis=1, keepdims=True)
    keptg = ((idx4 == g1) | (idx4 == g2)).astype(jnp.float32)
    kept = jnp.concatenate(
        [keptg[:, g:g + 1] for g in range(NG) for _ in range(GSZ)],
        axis=1)

    ms = jnp.where(kept > 0.5, s, 0.0)
    idx8 = lax.broadcasted_iota(jnp.int32, (T, E), 1)
    e1v = jnp.max(ms, axis=1, keepdims=True)
    i1 = jnp.min(jnp.where(ms == e1v, idx8, E), axis=1, keepdims=True)
    ms2 = jnp.where(idx8 == i1, neg, ms)
    e2v = jnp.max(ms2, axis=1, keepdims=True)
    i2 = jnp.min(jnp.where(ms2 == e2v, idx8, E), axis=1, keepdims=True)

    oh1 = (idx8 == i1).astype(jnp.float32)
    oh2 = (idx8 == i2).astype(jnp.float32)
    sel1 = jnp.sum(oh1 * sig, axis=1, keepdims=True)
    sel2 = jnp.sum(oh2 * sig, axis=1, keepdims=True)
    den = sel1 + sel2 + 1e-20
    w0_ref[...] = (sel1 / den * SCALE).reshape(T)
    w1_ref[...] = (sel2 / den * SCALE).reshape(T)

    a = oh1 + oh2
    c = a
    k = 1
    while k < T:
        c = c + jnp.concatenate(
            [jnp.zeros((k, E), jnp.float32), c[:T - k]], axis=0)
        k *= 2
    counts = c[T - 1:T, :]
    parts = [jnp.zeros((1, 1), jnp.float32)]
    run = jnp.zeros((1, 1), jnp.float32)
    for e in range(1, E):
        run = run + counts[:, e - 1:e]
        parts.append(run)
    off = jnp.concatenate(parts, axis=1)
    p0 = c - a
    p1 = c - oh2
    r0_ref[...] = jnp.sum(oh1 * (off + p0), axis=1).astype(jnp.int32)
    r1_ref[...] = jnp.sum(oh2 * (off + p1), axis=1).astype(jnp.int32)

    cnt_i = counts.astype(jnp.int32)
    off_i = off.astype(jnp.int32)
    offe_i = off_i + cnt_i
    first_blk = off_i // BT
    nt = jnp.where(cnt_i > 0, (offe_i - 1) // BT - first_blk + 1, 0)
    tparts = [jnp.zeros((1, 1), jnp.int32)]
    trun = jnp.zeros((1, 1), jnp.int32)
    for e in range(1, E):
        trun = trun + nt[:, e - 1:e]
        tparts.append(trun)
    tstart = jnp.concatenate(tparts, axis=1)
    tend = tstart + nt
    total = tend[:, E - 1:E]
    gcol = lax.broadcasted_iota(jnp.int32, (G, 1), 0)
    tendb = jnp.broadcast_to(tend, (G, E))
    eg = jnp.sum((tendb <= gcol).astype(jnp.int32), axis=1,
                 keepdims=True)
    egc = jnp.clip(eg, 0, E - 1)
    iotae = lax.broadcasted_iota(jnp.int32, (G, E), 1)
    sel = (iotae == egc).astype(jnp.int32)

    def pick(v):
        return jnp.sum(sel * jnp.broadcast_to(v, (G, E)), axis=1,
                       keepdims=True)

    valid = gcol < total
    te_last = jnp.sum((tend <= total - 1).astype(jnp.int32), axis=1,
                      keepdims=True)
    te_last = jnp.clip(te_last, 0, E - 1)
    te_ref[...] = jnp.where(valid, egc,
                            jnp.broadcast_to(te_last, (G, 1))).reshape(G)
    rb_ref[...] = jnp.where(valid, pick(first_blk) + (gcol - pick(tstart)),
                            NB - 1).reshape(G)
    lo_ref[...] = jnp.where(valid, pick(off_i), 0).reshape(G)
    hi_ref[...] = jnp.where(valid, pick(offe_i), 0).reshape(G)


def _router_tc(x, gate_w, gate_bias):
    return pl.pallas_call(
        _router_body,
        out_shape=(
            jax.ShapeDtypeStruct((T,), jnp.int32),
            jax.ShapeDtypeStruct((T,), jnp.int32),
            jax.ShapeDtypeStruct((T,), jnp.float32),
            jax.ShapeDtypeStruct((T,), jnp.float32),
            jax.ShapeDtypeStruct((G,), jnp.int32),
            jax.ShapeDtypeStruct((G,), jnp.int32),
            jax.ShapeDtypeStruct((G,), jnp.int32),
            jax.ShapeDtypeStruct((G,), jnp.int32),
        ),
    )(x, gate_w, gate_bias.reshape(1, E))


def _gmm_body(te_ref, rb_ref, lo_ref, hi_ref, xs_ref, wg_ref, wu_ref, wd_ref,
              out_ref):
    g = pl.program_id(0)

    @pl.when(hi_ref[g] > lo_ref[g])
    def _():
        xb = xs_ref[...].astype(jnp.float32)
        hg = lax.dot_general(xb, wg_ref[0], (((1,), (0,)), ((), ())),
                             preferred_element_type=jnp.float32)
        hu = lax.dot_general(xb, wu_ref[0], (((1,), (0,)), ((), ())),
                             preferred_element_type=jnp.float32)
        h = hg * jax.nn.sigmoid(hg) * hu
        y = lax.dot_general(h, wd_ref[0], (((1,), (0,)), ((), ())),
                            preferred_element_type=jnp.float32)
        rows = rb_ref[g] * BT + lax.broadcasted_iota(jnp.int32, (BT, 1), 0)
        mask = (rows >= lo_ref[g]) & (rows < hi_ref[g])
        out_ref[...] = jnp.where(mask, y, out_ref[...])


def _gmm_tc(te, rb, lo, hi, xs, w_gate, w_up, w_down):
    grid_spec = pltpu.PrefetchScalarGridSpec(
        num_scalar_prefetch=4,
        grid=(G,),
        in_specs=[
            pl.BlockSpec((BT, D), lambda g, te, rb, lo, hi: (rb[g], 0)),
            pl.BlockSpec((1, D, F), lambda g, te, rb, lo, hi: (te[g], 0, 0)),
            pl.BlockSpec((1, D, F), lambda g, te, rb, lo, hi: (te[g], 0, 0)),
            pl.BlockSpec((1, F, D), lambda g, te, rb, lo, hi: (te[g], 0, 0)),
        ],
        out_specs=pl.BlockSpec((BT, D), lambda g, te, rb, lo, hi: (rb[g], 0)),
    )
    return pl.pallas_call(
        _gmm_body,
        grid_spec=grid_spec,
        out_shape=jax.ShapeDtypeStruct((A, D), jnp.float32),
    )(te, rb, lo, hi, xs, w_gate, w_up, w_down)


def _shared_body(x_ref, wg_ref, wu_ref, wd_ref, out_ref):
    xb = x_ref[...]
    hg = lax.dot_general(xb, wg_ref[...], (((1,), (0,)), ((), ())),
                         preferred_element_type=jnp.float32)
    hu = lax.dot_general(xb, wu_ref[...], (((1,), (0,)), ((), ())),
                         preferred_element_type=jnp.float32)
    h = hg * jax.nn.sigmoid(hg) * hu
    out_ref[...] = lax.dot_general(h, wd_ref[...], (((1,), (0,)), ((), ())),
                                   preferred_element_type=jnp.float32)


def _shared_tc(x, sw_gate, sw_up, sw_down):
    sbt = 256
    return pl.pallas_call(
        _shared_body,
        grid=(T // sbt,),
        in_specs=[
            pl.BlockSpec((sbt, D), lambda i: (i, 0)),
            pl.BlockSpec((D, SF), lambda i: (0, 0)),
            pl.BlockSpec((D, SF), lambda i: (0, 0)),
            pl.BlockSpec((SF, D), lambda i: (0, 0)),
        ],
        out_specs=pl.BlockSpec((sbt, D), lambda i: (i, 0)),
        out_shape=jax.ShapeDtypeStruct((T, D), jnp.float32),
    )(x, sw_gate, sw_up, sw_down)


def _dispatch_body(x_hbm, r0_hbm, r1_hbm, xs_hbm, xbuf, i0, i1, sem):
    wid = lax.axis_index("s") * NC + lax.axis_index("c")
    base = wid * TPW
    pltpu.sync_copy(x_hbm.at[pl.ds(base, TPW)], xbuf)
    pltpu.sync_copy(r0_hbm.at[pl.ds(base, TPW)], i0)
    pltpu.sync_copy(r1_hbm.at[pl.ds(base, TPW)], i1)
    copies = []
    for c in range(TPW // CH):
        src = xbuf.at[pl.ds(c * CH, CH)]
        copies.append(
            pltpu.async_copy(src, xs_hbm.at[i0[pl.ds(c * CH, CH)]], sem))
        copies.append(
            pltpu.async_copy(src, xs_hbm.at[i1[pl.ds(c * CH, CH)]], sem))
    for cp in copies:
        cp.wait()


def _dispatch_sc(x, r0f, r1f):
    mesh = plsc.VectorSubcoreMesh(core_axis_name="c", subcore_axis_name="s")
    return pl.kernel(
        _dispatch_body,
        mesh=mesh,
        out_type=jax.ShapeDtypeStruct((A, D), jnp.float32),
        scratch_types=[
            pltpu.VMEM((TPW, D), jnp.float32),
            pltpu.VMEM((TPW,), jnp.int32),
            pltpu.VMEM((TPW,), jnp.int32),
            pltpu.SemaphoreType.DMA,
        ],
    )(x, r0f, r1f)


def _combine_body(ys_hbm, r0_hbm, r1_hbm, w0_hbm, w1_hbm, sh_hbm, out_hbm,
                  i0, i1, v0, v1, y0a, y0b, y1a, y1b, oba, obb,
                  sg0a, sg0b, sg1a, sg1b, ssha, sshb, ssta, sstb):
    wid = lax.axis_index("s") * NC + lax.axis_index("c")
    base = wid * TPW
    pltpu.sync_copy(r0_hbm.at[pl.ds(base, TPW)], i0)
    pltpu.sync_copy(r1_hbm.at[pl.ds(base, TPW)], i1)
    pltpu.sync_copy(w0_hbm.at[pl.ds(base, TPW)], v0)
    pltpu.sync_copy(w1_hbm.at[pl.ds(base, TPW)], v1)
    y0 = (y0a, y0b)
    y1 = (y1a, y1b)
    ob = (oba, obb)
    sg0 = (sg0a, sg0b)
    sg1 = (sg1a, sg1b)
    ssh = (ssha, sshb)
    sst = (ssta, sstb)
    nch = TPW // CH

    def fire(c):
        p = c & 1
        return (
            pltpu.async_copy(ys_hbm.at[i0[pl.ds(c * CH, CH)]], y0[p], sg0[p]),
            pltpu.async_copy(ys_hbm.at[i1[pl.ds(c * CH, CH)]], y1[p], sg1[p]),
            pltpu.async_copy(sh_hbm.at[pl.ds(base + c * CH, CH)], ob[p],
                             ssh[p]),
        )

    gh = {0: fire(0)}
    sth = {}
    for c in range(nch):
        p = c & 1
        for h in gh[c]:
            h.wait()
        if c + 1 < nch:
            if c >= 1:
                sth[c - 1].wait()
            gh[c + 1] = fire(c + 1)
        vv0 = v0[pl.ds(c * CH, CH)]
        vv1 = v1[pl.ds(c * CH, CH)]
        a0s = [vv0[t] for t in range(CH)]
        a1s = [vv1[t] for t in range(CH)]

        def body(v, carry, p=p, a0s=a0s, a1s=a1s):
            sl = pl.ds(v * 16, 16)
            for t in range(CH):
                ob[p][t, sl] = (ob[p][t, sl] + a0s[t] * y0[p][t, sl]
                                + a1s[t] * y1[p][t, sl])
            return carry

        lax.fori_loop(0, D // 16, body, 0)
        sth[c] = pltpu.async_copy(ob[p], out_hbm.at[pl.ds(base + c * CH, CH)],
                                  sst[p])
    sth[nch - 2].wait()
    sth[nch - 1].wait()


def _combine_sc(ys, r0f, r1f, w0f, w1f, shared):
    mesh = plsc.VectorSubcoreMesh(core_axis_name="c", subcore_axis_name="s")
    return pl.kernel(
        _combine_body,
        mesh=mesh,
        out_type=jax.ShapeDtypeStruct((T, D), jnp.float32),
        scratch_types=(
            [pltpu.VMEM((TPW,), jnp.int32)] * 2
            + [pltpu.VMEM((TPW,), jnp.float32)] * 2
            + [pltpu.VMEM((CH, D), jnp.float32)] * 6
            + [pltpu.SemaphoreType.DMA] * 8
        ),
    )(ys, r0f, r1f, w0f, w1f, shared)


def kernel(x, gate_w, gate_bias, w_gate, w_up, w_down, sw_gate, sw_up,
           sw_down):
    r0f, r1f, w0f, w1f, te, rb, lo, hi = _router_tc(x, gate_w, gate_bias)
    xs = _dispatch_sc(x, r0f, r1f)
    ys = _gmm_tc(te, rb, lo, hi, xs, w_gate, w_up, w_down)
    shared = _shared_tc(x, sw_gate, sw_up, sw_down)
    return _combine_sc(ys, r0f, r1f, w0f, w1f, shared)


# speedup vs baseline: 1.0407x; 1.0407x over previous
import jax
import jax.numpy as jnp
from jax import lax
from jax.experimental import pallas as pl
from jax.experimental.pallas import tpu as pltpu
from jax.experimental.pallas import tpu_sc as plsc

T = 2048
D = 1024
E = 8
F = 512
SF = 512
NG = 4
GSZ = E // NG
SCALE = 2.446
A = 2 * T

BT = 512
NB = A // BT
G = NB + E

NC = 2
NS = 16
NW = NC * NS
TPW = T // NW
CH = 16


def _router_body(x_ref, gw_ref, gb_ref, r0_ref, r1_ref, w0_ref, w1_ref,
                 te_ref, rb_ref, lo_ref, hi_ref):
    x = x_ref[...]
    gates = lax.dot_general(x, gw_ref[...], (((1,), (1,)), ((), ())),
                            preferred_element_type=jnp.float32)
    sig = jax.nn.sigmoid(gates)
    s = sig + gb_ref[...]

    gsum = jnp.concatenate(
        [s[:, g * GSZ:g * GSZ + 1] + s[:, g * GSZ + 1:g * GSZ + 2]
         for g in range(NG)], axis=1)

    neg = jnp.float32(-jnp.inf)
    idx4 = lax.broadcasted_iota(jnp.int32, (T, NG), 1)
    m1 = jnp.max(gsum, axis=1, keepdims=True)
    g1 = jnp.min(jnp.where(gsum == m1, idx4, NG), axis=1, keepdims=True)
    gs2 = jnp.where(idx4 == g1, neg, gsum)
    m2 = jnp.max(gs2, axis=1, keepdims=True)
    g2 = jnp.min(jnp.where(gs2 == m2, idx4, NG), axis=1, keepdims=True)
    keptg = ((idx4 == g1) | (idx4 == g2)).astype(jnp.float32)
    kept = jnp.concatenate(
        [keptg[:, g:g + 1] for g in range(NG) for _ in range(GSZ)],
        axis=1)

    ms = jnp.where(kept > 0.5, s, 0.0)
    idx8 = lax.broadcasted_iota(jnp.int32, (T, E), 1)
    e1v = jnp.max(ms, axis=1, keepdims=True)
    i1 = jnp.min(jnp.where(ms == e1v, idx8, E), axis=1, keepdims=True)
    ms2 = jnp.where(idx8 == i1, neg, ms)
    e2v = jnp.max(ms2, axis=1, keepdims=True)
    i2 = jnp.min(jnp.where(ms2 == e2v, idx8, E), axis=1, keepdims=True)

    oh1 = (idx8 == i1).astype(jnp.float32)
    oh2 = (idx8 == i2).astype(jnp.float32)
    sel1 = jnp.sum(oh1 * sig, axis=1, keepdims=True)
    sel2 = jnp.sum(oh2 * sig, axis=1, keepdims=True)
    den = sel1 + sel2 + 1e-20
    w0_ref[...] = (sel1 / den * SCALE).reshape(T)
    w1_ref[...] = (sel2 / den * SCALE).reshape(T)

    a = oh1 + oh2
    c = a
    k = 1
    while k < T:
        c = c + jnp.concatenate(
            [jnp.zeros((k, E), jnp.float32), c[:T - k]], axis=0)
        k *= 2
    counts = c[T - 1:T, :]
    parts = [jnp.zeros((1, 1), jnp.float32)]
    run = jnp.zeros((1, 1), jnp.float32)
    for e in range(1, E):
        run = run + counts[:, e - 1:e]
        parts.append(run)
    off = jnp.concatenate(parts, axis=1)
    p0 = c - a
    p1 = c - oh2
    r0_ref[...] = jnp.sum(oh1 * (off + p0), axis=1).astype(jnp.int32)
    r1_ref[...] = jnp.sum(oh2 * (off + p1), axis=1).astype(jnp.int32)

    cnt_i = counts.astype(jnp.int32)
    off_i = off.astype(jnp.int32)
    offe_i = off_i + cnt_i
    first_blk = off_i // BT
    nt = jnp.where(cnt_i > 0, (offe_i - 1) // BT - first_blk + 1, 0)
    tparts = [jnp.zeros((1, 1), jnp.int32)]
    trun = jnp.zeros((1, 1), jnp.int32)
    for e in range(1, E):
        trun = trun + nt[:, e - 1:e]
        tparts.append(trun)
    tstart = jnp.concatenate(tparts, axis=1)
    tend = tstart + nt
    total = tend[:, E - 1:E]
    gcol = lax.broadcasted_iota(jnp.int32, (G, 1), 0)
    tendb = jnp.broadcast_to(tend, (G, E))
    eg = jnp.sum((tendb <= gcol).astype(jnp.int32), axis=1,
                 keepdims=True)
    egc = jnp.clip(eg, 0, E - 1)
    iotae = lax.broadcasted_iota(jnp.int32, (G, E), 1)
    sel = (iotae == egc).astype(jnp.int32)

    def pick(v):
        return jnp.sum(sel * jnp.broadcast_to(v, (G, E)), axis=1,
                       keepdims=True)

    valid = gcol < total
    te_last = jnp.sum((tend <= total - 1).astype(jnp.int32), axis=1,
                      keepdims=True)
    te_last = jnp.clip(te_last, 0, E - 1)
    te_ref[...] = jnp.where(valid, egc,
                            jnp.broadcast_to(te_last, (G, 1))).reshape(G)
    rb_ref[...] = jnp.where(valid, pick(first_blk) + (gcol - pick(tstart)),
                            NB - 1).reshape(G)
    lo_ref[...] = jnp.where(valid, pick(off_i), 0).reshape(G)
    hi_ref[...] = jnp.where(valid, pick(offe_i), 0).reshape(G)


def _router_tc(x, gate_w, gate_bias):
    return pl.pallas_call(
        _router_body,
        out_shape=(
            jax.ShapeDtypeStruct((T,), jnp.int32),
            jax.ShapeDtypeStruct((T,), jnp.int32),
            jax.ShapeDtypeStruct((T,), jnp.float32),
            jax.ShapeDtypeStruct((T,), jnp.float32),
            jax.ShapeDtypeStruct((G,), jnp.int32),
            jax.ShapeDtypeStruct((G,), jnp.int32),
            jax.ShapeDtypeStruct((G,), jnp.int32),
            jax.ShapeDtypeStruct((G,), jnp.int32),
        ),
    )(x, gate_w, gate_bias.reshape(1, E))


def _gmm_body(te_ref, rb_ref, lo_ref, hi_ref, xs_ref, wg_ref, wu_ref, wd_ref,
              out_ref):
    g = pl.program_id(0)

    @pl.when(hi_ref[g] > lo_ref[g])
    def _():
        xb = xs_ref[...].astype(jnp.float32)
        hg = lax.dot_general(xb, wg_ref[0], (((1,), (0,)), ((), ())),
                             preferred_element_type=jnp.float32)
        hu = lax.dot_general(xb, wu_ref[0], (((1,), (0,)), ((), ())),
                             preferred_element_type=jnp.float32)
        h = hg * jax.nn.sigmoid(hg) * hu
        y = lax.dot_general(h, wd_ref[0], (((1,), (0,)), ((), ())),
                            preferred_element_type=jnp.float32)
        rows = rb_ref[g] * BT + lax.broadcasted_iota(jnp.int32, (BT, 1), 0)
        mask = (rows >= lo_ref[g]) & (rows < hi_ref[g])
        out_ref[...] = jnp.where(mask, y, out_ref[...])


def _gmm_tc(te, rb, lo, hi, xs, w_gate, w_up, w_down):
    grid_spec = pltpu.PrefetchScalarGridSpec(
        num_scalar_prefetch=4,
        grid=(G,),
        in_specs=[
            pl.BlockSpec((BT, D), lambda g, te, rb, lo, hi: (rb[g], 0)),
            pl.BlockSpec((1, D, F), lambda g, te, rb, lo, hi: (te[g], 0, 0)),
            pl.BlockSpec((1, D, F), lambda g, te, rb, lo, hi: (te[g], 0, 0)),
            pl.BlockSpec((1, F, D), lambda g, te, rb, lo, hi: (te[g], 0, 0)),
        ],
        out_specs=pl.BlockSpec((BT, D), lambda g, te, rb, lo, hi: (rb[g], 0)),
    )
    return pl.pallas_call(
        _gmm_body,
        grid_spec=grid_spec,
        out_shape=jax.ShapeDtypeStruct((A, D), jnp.float32),
    )(te, rb, lo, hi, xs, w_gate, w_up, w_down)


def _shared_body(x_ref, wg_ref, wu_ref, wd_ref, out_ref):
    xb = x_ref[...]
    hg = lax.dot_general(xb, wg_ref[...], (((1,), (0,)), ((), ())),
                         preferred_element_type=jnp.float32)
    hu = lax.dot_general(xb, wu_ref[...], (((1,), (0,)), ((), ())),
                         preferred_element_type=jnp.float32)
    h = hg * jax.nn.sigmoid(hg) * hu
    out_ref[...] = lax.dot_general(h, wd_ref[...], (((1,), (0,)), ((), ())),
                                   preferred_element_type=jnp.float32)


def _shared_tc(x, sw_gate, sw_up, sw_down):
    sbt = 256
    return pl.pallas_call(
        _shared_body,
        grid=(T // sbt,),
        in_specs=[
            pl.BlockSpec((sbt, D), lambda i: (i, 0)),
            pl.BlockSpec((D, SF), lambda i: (0, 0)),
            pl.BlockSpec((D, SF), lambda i: (0, 0)),
            pl.BlockSpec((SF, D), lambda i: (0, 0)),
        ],
        out_specs=pl.BlockSpec((sbt, D), lambda i: (i, 0)),
        out_shape=jax.ShapeDtypeStruct((T, D), jnp.float32),
    )(x, sw_gate, sw_up, sw_down)


def _dispatch_body(x_hbm, r0_hbm, r1_hbm, xs_hbm, xbuf, i0, i1, sem):
    wid = lax.axis_index("s") * NC + lax.axis_index("c")
    base = wid * TPW
    pltpu.sync_copy(x_hbm.at[pl.ds(base, TPW)], xbuf)
    pltpu.sync_copy(r0_hbm.at[pl.ds(base, TPW)], i0)
    pltpu.sync_copy(r1_hbm.at[pl.ds(base, TPW)], i1)
    copies = []
    for c in range(TPW // CH):
        src = xbuf.at[pl.ds(c * CH, CH)]
        copies.append(
            pltpu.async_copy(src, xs_hbm.at[i0[pl.ds(c * CH, CH)]], sem))
        copies.append(
            pltpu.async_copy(src, xs_hbm.at[i1[pl.ds(c * CH, CH)]], sem))
    for cp in copies:
        cp.wait()


def _dispatch_sc(x, r0f, r1f):
    mesh = plsc.VectorSubcoreMesh(core_axis_name="c", subcore_axis_name="s")
    return pl.kernel(
        _dispatch_body,
        mesh=mesh,
        out_type=jax.ShapeDtypeStruct((A, D), jnp.float32),
        scratch_types=[
            pltpu.VMEM((TPW, D), jnp.float32),
            pltpu.VMEM((TPW,), jnp.int32),
            pltpu.VMEM((TPW,), jnp.int32),
            pltpu.SemaphoreType.DMA,
        ],
    )(x, r0f, r1f)


def _combine_body(ys_hbm, r0_hbm, r1_hbm, w0_hbm, w1_hbm, sh_hbm, out_hbm,
                  i0, i1, v0, v1, y0a, y0b, y1a, y1b, oba, obb,
                  sg0a, sg0b, sg1a, sg1b, ssha, sshb, ssta, sstb):
    wid = lax.axis_index("s") * NC + lax.axis_index("c")
    base = wid * TPW
    pltpu.sync_copy(r0_hbm.at[pl.ds(base, TPW)], i0)
    pltpu.sync_copy(r1_hbm.at[pl.ds(base, TPW)], i1)
    pltpu.sync_copy(w0_hbm.at[pl.ds(base, TPW)], v0)
    pltpu.sync_copy(w1_hbm.at[pl.ds(base, TPW)], v1)
    y0 = (y0a, y0b)
    y1 = (y1a, y1b)
    ob = (oba, obb)
    sg0 = (sg0a, sg0b)
    sg1 = (sg1a, sg1b)
    ssh = (ssha, sshb)
    sst = (ssta, sstb)
    nch = TPW // CH

    def fire(c):
        p = c & 1
        return (
            pltpu.async_copy(ys_hbm.at[i0[pl.ds(c * CH, CH)]], y0[p], sg0[p]),
            pltpu.async_copy(ys_hbm.at[i1[pl.ds(c * CH, CH)]], y1[p], sg1[p]),
            pltpu.async_copy(sh_hbm.at[pl.ds(base + c * CH, CH)], ob[p],
                             ssh[p]),
        )

    gh = {0: fire(0)}
    sth = {}
    for c in range(nch):
        p = c & 1
        for h in gh[c]:
            h.wait()
        if c + 1 < nch:
            if c >= 1:
                sth[c - 1].wait()
            gh[c + 1] = fire(c + 1)
        vv0 = v0[pl.ds(c * CH, CH)]
        vv1 = v1[pl.ds(c * CH, CH)]
        a0s = [vv0[t] for t in range(CH)]
        a1s = [vv1[t] for t in range(CH)]

        def body(v, carry, p=p, a0s=a0s, a1s=a1s):
            sl = pl.ds(v * 16, 16)
            for t in range(CH):
                ob[p][t, sl] = (ob[p][t, sl] + a0s[t] * y0[p][t, sl]
                                + a1s[t] * y1[p][t, sl])
            return carry

        lax.fori_loop(0, D // 16, body, 0)
        sth[c] = pltpu.async_copy(ob[p], out_hbm.at[pl.ds(base + c * CH, CH)],
                                  sst[p])
    sth[nch - 2].wait()
    sth[nch - 1].wait()


def _combine_sc(ys, r0f, r1f, w0f, w1f, shared):
    mesh = plsc.VectorSubcoreMesh(core_axis_name="c", subcore_axis_name="s")
    return pl.kernel(
        _combine_body,
        mesh=mesh,
        out_type=jax.ShapeDtypeStruct((T, D), jnp.float32),
        scratch_types=(
            [pltpu.VMEM((TPW,), jnp.int32)] * 2
            + [pltpu.VMEM((TPW,), jnp.float32)] * 2
            + [pltpu.VMEM((CH, D), jnp.float32)] * 6
            + [pltpu.SemaphoreType.DMA] * 8
        ),
    )(ys, r0f, r1f, w0f, w1f, shared)


def kernel(x, gate_w, gate_bias, w_gate, w_up, w_down, sw_gate, sw_up,
           sw_down):
    r0f, r1f, w0f, w1f, te, rb, lo, hi = _router_tc(x, gate_w, gate_bias)
    xs = _dispatch_sc(x, r0f, r1f)
    ys = _gmm_tc(te, rb, lo, hi, xs, w_gate, w_up, w_down)
    shared = _shared_tc(x, sw_gate, sw_up, sw_down)
    return _combine_sc(ys, r0f, r1f, w0f, w1f, shared)


# speedup vs baseline: 1.0626x; 1.0211x over previous
import jax
import jax.numpy as jnp
from jax import lax
from jax.experimental import pallas as pl
from jax.experimental.pallas import tpu as pltpu
from jax.experimental.pallas import tpu_sc as plsc

T = 2048
D = 1024
E = 8
F = 512
SF = 512
NG = 4
GSZ = E // NG
SCALE = 2.446
A = 2 * T

BT = 512
NB = A // BT
G = NB + E

NC = 2
NS = 16
NW = NC * NS
TPW = T // NW
CH = 16


def _router_body(x_ref, gw_ref, gb_ref, r0_ref, r1_ref, w0_ref, w1_ref,
                 te_ref, rb_ref, lo_ref, hi_ref):
    x = x_ref[...]
    gates = lax.dot_general(x, gw_ref[...], (((1,), (1,)), ((), ())),
                            preferred_element_type=jnp.float32)
    sig = jax.nn.sigmoid(gates)
    s = sig + gb_ref[...]

    gsum = jnp.concatenate(
        [s[:, g * GSZ:g * GSZ + 1] + s[:, g * GSZ + 1:g * GSZ + 2]
         for g in range(NG)], axis=1)

    neg = jnp.float32(-jnp.inf)
    idx4 = lax.broadcasted_iota(jnp.int32, (T, NG), 1)
    m1 = jnp.max(gsum, axis=1, keepdims=True)
    g1 = jnp.min(jnp.where(gsum == m1, idx4, NG), axis=1, keepdims=True)
    gs2 = jnp.where(idx4 == g1, neg, gsum)
    m2 = jnp.max(gs2, axis=1, keepdims=True)
    g2 = jnp.min(jnp.where(gs2 == m2, idx4, NG), axis=1, keepdims=True)
    keptg = ((idx4 == g1) | (idx4 == g2)).astype(jnp.float32)
    kept = jnp.concatenate(
        [keptg[:, g:g + 1] for g in range(NG) for _ in range(GSZ)],
        axis=1)

    ms = jnp.where(kept > 0.5, s, 0.0)
    idx8 = lax.broadcasted_iota(jnp.int32, (T, E), 1)
    e1v = jnp.max(ms, axis=1, keepdims=True)
    i1 = jnp.min(jnp.where(ms == e1v, idx8, E), axis=1, keepdims=True)
    ms2 = jnp.where(idx8 == i1, neg, ms)
    e2v = jnp.max(ms2, axis=1, keepdims=True)
    i2 = jnp.min(jnp.where(ms2 == e2v, idx8, E), axis=1, keepdims=True)

    oh1 = (idx8 == i1).astype(jnp.float32)
    oh2 = (idx8 == i2).astype(jnp.float32)
    sel1 = jnp.sum(oh1 * sig, axis=1, keepdims=True)
    sel2 = jnp.sum(oh2 * sig, axis=1, keepdims=True)
    den = sel1 + sel2 + 1e-20
    w0_ref[...] = (sel1 / den * SCALE).reshape(T)
    w1_ref[...] = (sel2 / den * SCALE).reshape(T)

    a = oh1 + oh2
    c = a
    k = 1
    while k < T:
        c = c + jnp.concatenate(
            [jnp.zeros((k, E), jnp.float32), c[:T - k]], axis=0)
        k *= 2
    counts = c[T - 1:T, :]
    parts = [jnp.zeros((1, 1), jnp.float32)]
    run = jnp.zeros((1, 1), jnp.float32)
    for e in range(1, E):
        run = run + counts[:, e - 1:e]
        parts.append(run)
    off = jnp.concatenate(parts, axis=1)
    p0 = c - a
    p1 = c - oh2
    r0_ref[...] = jnp.sum(oh1 * (off + p0), axis=1).astype(jnp.int32)
    r1_ref[...] = jnp.sum(oh2 * (off + p1), axis=1).astype(jnp.int32)

    cnt_i = counts.astype(jnp.int32)
    off_i = off.astype(jnp.int32)
    offe_i = off_i + cnt_i
    first_blk = off_i // BT
    nt = jnp.where(cnt_i > 0, (offe_i - 1) // BT - first_blk + 1, 0)
    tparts = [jnp.zeros((1, 1), jnp.int32)]
    trun = jnp.zeros((1, 1), jnp.int32)
    for e in range(1, E):
        trun = trun + nt[:, e - 1:e]
        tparts.append(trun)
    tstart = jnp.concatenate(tparts, axis=1)
    tend = tstart + nt
    total = tend[:, E - 1:E]
    gcol = lax.broadcasted_iota(jnp.int32, (G, 1), 0)
    tendb = jnp.broadcast_to(tend, (G, E))
    eg = jnp.sum((tendb <= gcol).astype(jnp.int32), axis=1,
                 keepdims=True)
    egc = jnp.clip(eg, 0, E - 1)
    iotae = lax.broadcasted_iota(jnp.int32, (G, E), 1)
    sel = (iotae == egc).astype(jnp.int32)

    def pick(v):
        return jnp.sum(sel * jnp.broadcast_to(v, (G, E)), axis=1,
                       keepdims=True)

    valid = gcol < total
    te_last = jnp.sum((tend <= total - 1).astype(jnp.int32), axis=1,
                      keepdims=True)
    te_last = jnp.clip(te_last, 0, E - 1)
    te_ref[...] = jnp.where(valid, egc,
                            jnp.broadcast_to(te_last, (G, 1))).reshape(G)
    rb_ref[...] = jnp.where(valid, pick(first_blk) + (gcol - pick(tstart)),
                            NB - 1).reshape(G)
    lo_ref[...] = jnp.where(valid, pick(off_i), 0).reshape(G)
    hi_ref[...] = jnp.where(valid, pick(offe_i), 0).reshape(G)


def _router_tc(x, gate_w, gate_bias):
    return pl.pallas_call(
        _router_body,
        out_shape=(
            jax.ShapeDtypeStruct((T,), jnp.int32),
            jax.ShapeDtypeStruct((T,), jnp.int32),
            jax.ShapeDtypeStruct((T,), jnp.float32),
            jax.ShapeDtypeStruct((T,), jnp.float32),
            jax.ShapeDtypeStruct((G,), jnp.int32),
            jax.ShapeDtypeStruct((G,), jnp.int32),
            jax.ShapeDtypeStruct((G,), jnp.int32),
            jax.ShapeDtypeStruct((G,), jnp.int32),
        ),
    )(x, gate_w, gate_bias.reshape(1, E))


def _gmm_body(te_ref, rb_ref, lo_ref, hi_ref, xs_ref, wg_ref, wu_ref, wd_ref,
              out_ref):
    g = pl.program_id(0)

    @pl.when(hi_ref[g] > lo_ref[g])
    def _():
        xb = xs_ref[...].astype(jnp.float32)
        hg = lax.dot_general(xb, wg_ref[0], (((1,), (0,)), ((), ())),
                             preferred_element_type=jnp.float32)
        hu = lax.dot_general(xb, wu_ref[0], (((1,), (0,)), ((), ())),
                             preferred_element_type=jnp.float32)
        h = hg * jax.nn.sigmoid(hg) * hu
        y = lax.dot_general(h, wd_ref[0], (((1,), (0,)), ((), ())),
                            preferred_element_type=jnp.float32)
        rows = rb_ref[g] * BT + lax.broadcasted_iota(jnp.int32, (BT, 1), 0)
        mask = (rows >= lo_ref[g]) & (rows < hi_ref[g])
        out_ref[...] = jnp.where(mask, y, out_ref[...])


def _gmm_tc(te, rb, lo, hi, xs, w_gate, w_up, w_down):
    grid_spec = pltpu.PrefetchScalarGridSpec(
        num_scalar_prefetch=4,
        grid=(G,),
        in_specs=[
            pl.BlockSpec((BT, D), lambda g, te, rb, lo, hi: (rb[g], 0)),
            pl.BlockSpec((1, D, F), lambda g, te, rb, lo, hi: (te[g], 0, 0)),
            pl.BlockSpec((1, D, F), lambda g, te, rb, lo, hi: (te[g], 0, 0)),
            pl.BlockSpec((1, F, D), lambda g, te, rb, lo, hi: (te[g], 0, 0)),
        ],
        out_specs=pl.BlockSpec((BT, D), lambda g, te, rb, lo, hi: (rb[g], 0)),
    )
    return pl.pallas_call(
        _gmm_body,
        grid_spec=grid_spec,
        out_shape=jax.ShapeDtypeStruct((A, D), jnp.float32),
    )(te, rb, lo, hi, xs, w_gate, w_up, w_down)


def _shared_body(x_ref, wg_ref, wu_ref, wd_ref, out_ref):
    xb = x_ref[...]
    hg = lax.dot_general(xb, wg_ref[...], (((1,), (0,)), ((), ())),
                         preferred_element_type=jnp.float32)
    hu = lax.dot_general(xb, wu_ref[...], (((1,), (0,)), ((), ())),
                         preferred_element_type=jnp.float32)
    h = hg * jax.nn.sigmoid(hg) * hu
    out_ref[...] = lax.dot_general(h, wd_ref[...], (((1,), (0,)), ((), ())),
                                   preferred_element_type=jnp.float32)


def _shared_tc(x, sw_gate, sw_up, sw_down):
    sbt = 512
    return pl.pallas_call(
        _shared_body,
        grid=(T // sbt,),
        in_specs=[
            pl.BlockSpec((sbt, D), lambda i: (i, 0)),
            pl.BlockSpec((D, SF), lambda i: (0, 0)),
            pl.BlockSpec((D, SF), lambda i: (0, 0)),
            pl.BlockSpec((SF, D), lambda i: (0, 0)),
        ],
        out_specs=pl.BlockSpec((sbt, D), lambda i: (i, 0)),
        out_shape=jax.ShapeDtypeStruct((T, D), jnp.float32),
    )(x, sw_gate, sw_up, sw_down)


def _dispatch_body(x_hbm, r0_hbm, r1_hbm, xs_hbm, xbuf, i0, i1, sem):
    wid = lax.axis_index("s") * NC + lax.axis_index("c")
    base = wid * TPW
    pltpu.sync_copy(x_hbm.at[pl.ds(base, TPW)], xbuf)
    pltpu.sync_copy(r0_hbm.at[pl.ds(base, TPW)], i0)
    pltpu.sync_copy(r1_hbm.at[pl.ds(base, TPW)], i1)
    copies = []
    for c in range(TPW // CH):
        src = xbuf.at[pl.ds(c * CH, CH)]
        copies.append(
            pltpu.async_copy(src, xs_hbm.at[i0[pl.ds(c * CH, CH)]], sem))
        copies.append(
            pltpu.async_copy(src, xs_hbm.at[i1[pl.ds(c * CH, CH)]], sem))
    for cp in copies:
        cp.wait()


def _dispatch_sc(x, r0f, r1f):
    mesh = plsc.VectorSubcoreMesh(core_axis_name="c", subcore_axis_name="s")
    return pl.kernel(
        _dispatch_body,
        mesh=mesh,
        out_type=jax.ShapeDtypeStruct((A, D), jnp.float32),
        scratch_types=[
            pltpu.VMEM((TPW, D), jnp.float32),
            pltpu.VMEM((TPW,), jnp.int32),
            pltpu.VMEM((TPW,), jnp.int32),
            pltpu.SemaphoreType.DMA,
        ],
    )(x, r0f, r1f)


def _combine_body(ys_hbm, r0_hbm, r1_hbm, w0_hbm, w1_hbm, sh_hbm, out_hbm,
                  i0, i1, v0, v1, y0a, y0b, y1a, y1b, oba, obb,
                  sg0a, sg0b, sg1a, sg1b, ssha, sshb, ssta, sstb):
    wid = lax.axis_index("s") * NC + lax.axis_index("c")
    base = wid * TPW
    pltpu.sync_copy(r0_hbm.at[pl.ds(base, TPW)], i0)
    pltpu.sync_copy(r1_hbm.at[pl.ds(base, TPW)], i1)
    pltpu.sync_copy(w0_hbm.at[pl.ds(base, TPW)], v0)
    pltpu.sync_copy(w1_hbm.at[pl.ds(base, TPW)], v1)
    y0 = (y0a, y0b)
    y1 = (y1a, y1b)
    ob = (oba, obb)
    sg0 = (sg0a, sg0b)
    sg1 = (sg1a, sg1b)
    ssh = (ssha, sshb)
    sst = (ssta, sstb)
    nch = TPW // CH

    def fire(c):
        p = c & 1
        return (
            pltpu.async_copy(ys_hbm.at[i0[pl.ds(c * CH, CH)]], y0[p], sg0[p]),
            pltpu.async_copy(ys_hbm.at[i1[pl.ds(c * CH, CH)]], y1[p], sg1[p]),
            pltpu.async_copy(sh_hbm.at[pl.ds(base + c * CH, CH)], ob[p],
                             ssh[p]),
        )

    gh = {0: fire(0)}
    sth = {}
    for c in range(nch):
        p = c & 1
        for h in gh[c]:
            h.wait()
        if c + 1 < nch:
            if c >= 1:
                sth[c - 1].wait()
            gh[c + 1] = fire(c + 1)
        vv0 = v0[pl.ds(c * CH, CH)]
        vv1 = v1[pl.ds(c * CH, CH)]
        a0s = [vv0[t] for t in range(CH)]
        a1s = [vv1[t] for t in range(CH)]

        def body(v, carry, p=p, a0s=a0s, a1s=a1s):
            sl = pl.ds(v * 16, 16)
            for t in range(CH):
                ob[p][t, sl] = (ob[p][t, sl] + a0s[t] * y0[p][t, sl]
                                + a1s[t] * y1[p][t, sl])
            return carry

        lax.fori_loop(0, D // 16, body, 0)
        sth[c] = pltpu.async_copy(ob[p], out_hbm.at[pl.ds(base + c * CH, CH)],
                                  sst[p])
    sth[nch - 2].wait()
    sth[nch - 1].wait()


def _combine_sc(ys, r0f, r1f, w0f, w1f, shared):
    mesh = plsc.VectorSubcoreMesh(core_axis_name="c", subcore_axis_name="s")
    return pl.kernel(
        _combine_body,
        mesh=mesh,
        out_type=jax.ShapeDtypeStruct((T, D), jnp.float32),
        scratch_types=(
            [pltpu.VMEM((TPW,), jnp.int32)] * 2
            + [pltpu.VMEM((TPW,), jnp.float32)] * 2
            + [pltpu.VMEM((CH, D), jnp.float32)] * 6
            + [pltpu.SemaphoreType.DMA] * 8
        ),
    )(ys, r0f, r1f, w0f, w1f, shared)


def kernel(x, gate_w, gate_bias, w_gate, w_up, w_down, sw_gate, sw_up,
           sw_down):
    r0f, r1f, w0f, w1f, te, rb, lo, hi = _router_tc(x, gate_w, gate_bias)
    xs = _dispatch_sc(x, r0f, r1f)
    ys = _gmm_tc(te, rb, lo, hi, xs, w_gate, w_up, w_down)
    shared = _shared_tc(x, sw_gate, sw_up, sw_down)
    return _combine_sc(ys, r0f, r1f, w0f, w1f, shared)
